# Initial kernel scaffold; baseline (speedup 1.0000x reference)
import functools

import jax
import jax.numpy as jnp
from jax import lax
from jax.experimental import pallas as pl
from jax.experimental.pallas import tpu as pltpu
from jax.experimental.pallas import tpu_sc as plsc

N_NODES = 10000
N_EDGES = 320000
N_FEATURES = 128
N_HIDDEN = 64
N_CLASSES = 16
K = 10
ALPHA = 0.1

NS = 16
LANES = 16
C = 128
NCHUNK = 157
EPT = NCHUNK * C
E_PAD = EPT * NS
NP = 10016
RPT = NP // NS


def _mlp_body(x_ref, w1t_ref, w2t_ref, h_ref):
    h = jnp.maximum(jnp.dot(x_ref[...], w1t_ref[...],
                            preferred_element_type=jnp.float32), 0.0)
    h_ref[...] = jnp.dot(h, w2t_ref[...], preferred_element_type=jnp.float32)


def _mlp(xp, w1t, w2t):
    blk = NP // 4
    return pl.pallas_call(
        _mlp_body,
        grid=(4,),
        in_specs=[
            pl.BlockSpec((blk, N_FEATURES), lambda i: (i, 0)),
            pl.BlockSpec((N_FEATURES, N_HIDDEN), lambda i: (0, 0)),
            pl.BlockSpec((N_HIDDEN, N_CLASSES), lambda i: (0, 0)),
        ],
        out_specs=pl.BlockSpec((blk, N_CLASSES), lambda i: (i, 0)),
        out_shape=jax.ShapeDtypeStruct((NP, N_CLASSES), jnp.float32),
    )(xp, w1t, w2t)


_MESH = plsc.VectorSubcoreMesh(
    core_axis_name="c", subcore_axis_name="s", num_cores=1)


@functools.partial(
    pl.kernel,
    out_type=jax.ShapeDtypeStruct((NP, LANES), jnp.float32),
    mesh=_MESH,
    scratch_types=[
        pltpu.VMEM_SHARED((NP, LANES), jnp.float32),
        pltpu.VMEM_SHARED((NP, LANES), jnp.float32),
        pltpu.VMEM((NCHUNK, C), jnp.int32),
        pltpu.VMEM((NCHUNK, C), jnp.int32),
        pltpu.VMEM((RPT, LANES), jnp.float32),
        pltpu.VMEM((RPT, LANES), jnp.float32),
        pltpu.VMEM((RPT, LANES), jnp.float32),
        pltpu.VMEM((RPT, LANES), jnp.float32),
        pltpu.VMEM((RPT, LANES), jnp.float32),
        pltpu.VMEM((RPT, LANES), jnp.float32),
        pltpu.VMEM((C, LANES), jnp.float32),
        pltpu.SemaphoreType.DMA,
    ],
)
def _appnp_sc(src_hbm, dst_hbm, h_hbm, out_hbm, u_sh, acc_sh, srcix, dstix,
              u_buf, c1_buf, c2_buf, dv_buf, t_buf, z_buf, gbuf, gsem):
    t = lax.axis_index("s")
    base = t * RPT

    pltpu.sync_copy(src_hbm.at[t], srcix)
    pltpu.sync_copy(dst_hbm.at[t], dstix)

    def zfill(j, _):
        z_buf[j] = jnp.zeros((LANES,), jnp.float32)
        return 0
    lax.fori_loop(0, RPT, zfill, 0)

    def onesfill(j, _):
        gbuf[j] = jnp.full((LANES,), 1.0, jnp.float32)
        return 0
    lax.fori_loop(0, C, onesfill, 0)

    pltpu.sync_copy(z_buf, u_sh.at[pl.ds(base, RPT)])
    pltpu.sync_copy(z_buf, acc_sh.at[pl.ds(base, RPT)])
    plsc.subcore_barrier()

    def deg_body(j, _):
        pltpu.sync_copy(gbuf, acc_sh.at[dstix.at[j]], add=True)
        return 0
    lax.fori_loop(0, NCHUNK, deg_body, 0)
    plsc.subcore_barrier()

    pltpu.sync_copy(acc_sh.at[pl.ds(base, RPT)], t_buf)
    pltpu.sync_copy(h_hbm.at[pl.ds(base, RPT)], c2_buf)

    def init_body(j, _):
        deg = t_buf[j] + 1.0
        i = plsc.bitcast(deg, jnp.int32)
        i = jnp.int32(0x5F3759DF) - lax.shift_right_logical(i, 1)
        y = plsc.bitcast(i, jnp.float32)
        half = 0.5 * deg
        y = y * (1.5 - half * y * y)
        y = y * (1.5 - half * y * y)
        y = y * (1.5 - half * y * y)
        y = y * (1.5 - half * y * y)
        u0 = y * c2_buf[j]
        u_buf[j] = u0
        c2_buf[j] = ALPHA * u0
        c1_buf[j] = (1.0 - ALPHA) * y * y
        dv_buf[j] = y
        return 0
    lax.fori_loop(0, RPT, init_body, 0)

    pltpu.sync_copy(u_buf, u_sh.at[pl.ds(base, RPT)])
    pltpu.sync_copy(z_buf, acc_sh.at[pl.ds(base, RPT)])
    plsc.subcore_barrier()

    def step(_, carry):
        def edge_body(j, _c):
            pltpu.async_copy(u_sh.at[srcix.at[j]], gbuf, gsem).wait()
            pltpu.sync_copy(gbuf, acc_sh.at[dstix.at[j]], add=True)
            return 0
        lax.fori_loop(0, NCHUNK, edge_body, 0)
        plsc.subcore_barrier()

        pltpu.sync_copy(acc_sh.at[pl.ds(base, RPT)], t_buf)

        def comb(j, _c):
            u_buf[j] = c1_buf[j] * (t_buf[j] + u_buf[j]) + c2_buf[j]
            return 0
        lax.fori_loop(0, RPT, comb, 0)

        pltpu.sync_copy(u_buf, u_sh.at[pl.ds(base, RPT)])
        pltpu.sync_copy(z_buf, acc_sh.at[pl.ds(base, RPT)])
        plsc.subcore_barrier()
        return carry
    lax.fori_loop(0, K, step, 0)

    def fin(j, _c):
        t_buf[j] = u_buf[j] / dv_buf[j]
        return 0
    lax.fori_loop(0, RPT, fin, 0)
    pltpu.sync_copy(t_buf, out_hbm.at[pl.ds(base, RPT)])


def kernel(x, edge_index, W1, W2):
    src = edge_index[0].astype(jnp.int32)
    dst = edge_index[1].astype(jnp.int32)
    pad = E_PAD - N_EDGES
    pad_idx = N_NODES + (jnp.arange(pad, dtype=jnp.int32) % 16)
    srcp = jnp.concatenate([src, pad_idx]).reshape(NS, NCHUNK, C)
    dstp = jnp.concatenate([dst, pad_idx]).reshape(NS, NCHUNK, C)

    xp = jnp.pad(x, ((0, NP - N_NODES), (0, 0)))
    h = _mlp(xp, W1.T, W2.T)
    z = _appnp_sc(srcp, dstp, h)
    return z[:N_NODES]


# speedup vs baseline: 38.3673x; 38.3673x over previous
import functools

import jax
import jax.numpy as jnp
from jax import lax
from jax.experimental import pallas as pl
from jax.experimental.pallas import tpu as pltpu
from jax.experimental.pallas import tpu_sc as plsc

N_NODES = 10000
N_EDGES = 320000
N_FEATURES = 128
N_HIDDEN = 64
N_CLASSES = 16
K = 10
ALPHA = 0.1

NS = 16
LANES = 16
C = 128
NCHUNK = 157
EPT = NCHUNK * C
E_PAD = EPT * NS
NP = 10112
RPT = NP // NS


def _mlp_body(x_ref, w1t_ref, w2t_ref, h_ref):
    h = jnp.maximum(jnp.dot(x_ref[...], w1t_ref[...],
                            preferred_element_type=jnp.float32), 0.0)
    h_ref[...] = jnp.dot(h, w2t_ref[...], preferred_element_type=jnp.float32)


def _mlp(xp, w1t, w2t):
    blk = NP // 4
    return pl.pallas_call(
        _mlp_body,
        grid=(4,),
        in_specs=[
            pl.BlockSpec((blk, N_FEATURES), lambda i: (i, 0)),
            pl.BlockSpec((N_FEATURES, N_HIDDEN), lambda i: (0, 0)),
            pl.BlockSpec((N_HIDDEN, N_CLASSES), lambda i: (0, 0)),
        ],
        out_specs=pl.BlockSpec((blk, N_CLASSES), lambda i: (i, 0)),
        out_shape=jax.ShapeDtypeStruct((NP, N_CLASSES), jnp.float32),
    )(xp, w1t, w2t)


_MESH = plsc.VectorSubcoreMesh(
    core_axis_name="c", subcore_axis_name="s", num_cores=1)


@functools.partial(
    pl.kernel,
    out_type=jax.ShapeDtypeStruct((NP, LANES), jnp.float32),
    mesh=_MESH,
    compiler_params=pltpu.CompilerParams(use_tc_tiling_on_sc=False),
    scratch_types=[
        pltpu.VMEM_SHARED((NP, LANES), jnp.float32),
        pltpu.VMEM_SHARED((NP, LANES), jnp.float32),
        pltpu.VMEM((NCHUNK, C), jnp.int32),
        pltpu.VMEM((NCHUNK, C), jnp.int32),
        pltpu.VMEM((RPT, LANES), jnp.float32),
        pltpu.VMEM((RPT, LANES), jnp.float32),
        pltpu.VMEM((RPT, LANES), jnp.float32),
        pltpu.VMEM((RPT, LANES), jnp.float32),
        pltpu.VMEM((RPT, LANES), jnp.float32),
        pltpu.VMEM((RPT, LANES), jnp.float32),
        pltpu.VMEM((C, LANES), jnp.float32),
        pltpu.SemaphoreType.DMA,
    ],
)
def _appnp_sc(src_hbm, dst_hbm, h_hbm, out_hbm, u_sh, acc_sh, srcix, dstix,
              u_buf, c1_buf, c2_buf, dv_buf, t_buf, z_buf, gbuf, gsem):
    t = lax.axis_index("s")
    base = pl.multiple_of(t * RPT, 8)

    pltpu.sync_copy(src_hbm.at[t], srcix)
    pltpu.sync_copy(dst_hbm.at[t], dstix)

    def zfill(j, _):
        z_buf[j] = jnp.zeros((LANES,), jnp.float32)
        return 0
    lax.fori_loop(0, RPT, zfill, 0)

    def onesfill(j, _):
        gbuf[j] = jnp.full((LANES,), 1.0, jnp.float32)
        return 0
    lax.fori_loop(0, C, onesfill, 0)

    pltpu.sync_copy(z_buf, u_sh.at[pl.ds(base, RPT)])
    pltpu.sync_copy(z_buf, acc_sh.at[pl.ds(base, RPT)])
    plsc.subcore_barrier()

    def deg_body(j, _):
        pltpu.sync_copy(gbuf, acc_sh.at[dstix.at[j]], add=True)
        return 0
    lax.fori_loop(0, NCHUNK, deg_body, 0)
    plsc.subcore_barrier()

    pltpu.sync_copy(acc_sh.at[pl.ds(base, RPT)], t_buf)
    pltpu.sync_copy(h_hbm.at[pl.ds(base, RPT)], c2_buf)

    def init_body(j, _):
        deg = t_buf[j] + 1.0
        i = lax.bitcast_convert_type(deg, jnp.int32)
        i = jnp.int32(0x5F3759DF) - lax.shift_right_logical(i, 1)
        y = lax.bitcast_convert_type(i, jnp.float32)
        half = 0.5 * deg
        y = y * (1.5 - half * y * y)
        y = y * (1.5 - half * y * y)
        y = y * (1.5 - half * y * y)
        y = y * (1.5 - half * y * y)
        u0 = y * c2_buf[j]
        u_buf[j] = u0
        c2_buf[j] = ALPHA * u0
        c1_buf[j] = (1.0 - ALPHA) * y * y
        dv_buf[j] = y
        return 0
    lax.fori_loop(0, RPT, init_body, 0)

    pltpu.sync_copy(u_buf, u_sh.at[pl.ds(base, RPT)])
    pltpu.sync_copy(z_buf, acc_sh.at[pl.ds(base, RPT)])
    plsc.subcore_barrier()

    def step(_, carry):
        def edge_body(j, _c):
            pltpu.async_copy(u_sh.at[srcix.at[j]], gbuf, gsem).wait()
            pltpu.sync_copy(gbuf, acc_sh.at[dstix.at[j]], add=True)
            return 0
        lax.fori_loop(0, NCHUNK, edge_body, 0)
        plsc.subcore_barrier()

        pltpu.sync_copy(acc_sh.at[pl.ds(base, RPT)], t_buf)

        def comb(j, _c):
            u_buf[j] = c1_buf[j] * (t_buf[j] + u_buf[j]) + c2_buf[j]
            return 0
        lax.fori_loop(0, RPT, comb, 0)

        pltpu.sync_copy(u_buf, u_sh.at[pl.ds(base, RPT)])
        pltpu.sync_copy(z_buf, acc_sh.at[pl.ds(base, RPT)])
        plsc.subcore_barrier()
        return carry
    lax.fori_loop(0, K, step, 0)

    def fin(j, _c):
        t_buf[j] = u_buf[j] / dv_buf[j]
        return 0
    lax.fori_loop(0, RPT, fin, 0)
    pltpu.sync_copy(t_buf, out_hbm.at[pl.ds(base, RPT)])


def kernel(x, edge_index, W1, W2):
    src = edge_index[0].astype(jnp.int32)
    dst = edge_index[1].astype(jnp.int32)
    pad = E_PAD - N_EDGES
    pad_idx = N_NODES + (jnp.arange(pad, dtype=jnp.int32) % 16)
    srcp = jnp.concatenate([src, pad_idx]).reshape(NS, NCHUNK, C)
    dstp = jnp.concatenate([dst, pad_idx]).reshape(NS, NCHUNK, C)

    xp = jnp.pad(x, ((0, NP - N_NODES), (0, 0)))
    h = _mlp(xp, W1.T, W2.T)
    z = _appnp_sc(srcp, dstp, h)
    return z[:N_NODES]


# speedup vs baseline: 50.0090x; 1.3034x over previous
import functools

import jax
import jax.numpy as jnp
from jax import lax
from jax.experimental import pallas as pl
from jax.experimental.pallas import tpu as pltpu
from jax.experimental.pallas import tpu_sc as plsc

N_NODES = 10000
N_EDGES = 320000
N_FEATURES = 128
N_HIDDEN = 64
N_CLASSES = 16
K = 10
ALPHA = 0.1

NS = 16
LANES = 16
C = 128
NBUF = 8
NCHUNK = 160
NGRP = NCHUNK // NBUF
EPT = NCHUNK * C
E_PAD = EPT * NS
NP = 10112
RPT = NP // NS


def _mlp_body(x_ref, w1t_ref, w2t_ref, h_ref):
    h = jnp.maximum(jnp.dot(x_ref[...], w1t_ref[...],
                            preferred_element_type=jnp.float32), 0.0)
    h_ref[...] = jnp.dot(h, w2t_ref[...], preferred_element_type=jnp.float32)


def _mlp(xp, w1t, w2t):
    blk = NP // 4
    return pl.pallas_call(
        _mlp_body,
        grid=(4,),
        in_specs=[
            pl.BlockSpec((blk, N_FEATURES), lambda i: (i, 0)),
            pl.BlockSpec((N_FEATURES, N_HIDDEN), lambda i: (0, 0)),
            pl.BlockSpec((N_HIDDEN, N_CLASSES), lambda i: (0, 0)),
        ],
        out_specs=pl.BlockSpec((blk, N_CLASSES), lambda i: (i, 0)),
        out_shape=jax.ShapeDtypeStruct((NP, N_CLASSES), jnp.float32),
    )(xp, w1t, w2t)


_MESH = plsc.VectorSubcoreMesh(
    core_axis_name="c", subcore_axis_name="s", num_cores=1)


@functools.partial(
    pl.kernel,
    out_type=jax.ShapeDtypeStruct((NP, LANES), jnp.float32),
    mesh=_MESH,
    compiler_params=pltpu.CompilerParams(use_tc_tiling_on_sc=False),
    scratch_types=[
        pltpu.VMEM_SHARED((NP, LANES), jnp.float32),
        pltpu.VMEM_SHARED((NP, LANES), jnp.float32),
        pltpu.VMEM((NCHUNK, C), jnp.int32),
        pltpu.VMEM((NCHUNK, C), jnp.int32),
        pltpu.VMEM((RPT, LANES), jnp.float32),
        pltpu.VMEM((RPT, LANES), jnp.float32),
        pltpu.VMEM((RPT, LANES), jnp.float32),
        pltpu.VMEM((RPT, LANES), jnp.float32),
        pltpu.VMEM((RPT, LANES), jnp.float32),
        pltpu.VMEM((NBUF, C, LANES), jnp.float32),
        pltpu.SemaphoreType.DMA((NBUF,)),
        pltpu.SemaphoreType.DMA((NBUF,)),
    ],
)
def _appnp_sc(src_hbm, dst_hbm, h_hbm, out_hbm, u_sh, acc_sh, srcix, dstix,
              u_buf, c1_buf, c2_buf, t_buf, z_buf, gbufs, gsems, ssems):
    t = lax.axis_index("s")
    base = pl.multiple_of(t * RPT, 8)

    pltpu.sync_copy(src_hbm.at[t], srcix)
    pltpu.sync_copy(dst_hbm.at[t], dstix)

    def zfill(j, _):
        z_buf[j] = jnp.zeros((LANES,), jnp.float32)
        return 0
    lax.fori_loop(0, RPT, zfill, 0)

    def onesfill(j, _):
        gbufs[0, j] = jnp.full((LANES,), 1.0, jnp.float32)
        return 0
    lax.fori_loop(0, C, onesfill, 0)

    pltpu.sync_copy(z_buf, u_sh.at[pl.ds(base, RPT)])
    pltpu.sync_copy(z_buf, acc_sh.at[pl.ds(base, RPT)])
    plsc.subcore_barrier()

    def deg_group(jo, _c):
        j0 = jo * NBUF
        for b in range(NBUF):
            pltpu.async_copy(gbufs.at[0], acc_sh.at[dstix.at[j0 + b]],
                             ssems.at[b], add=True)
        for b in range(NBUF):
            pltpu.make_async_copy(gbufs.at[0], acc_sh.at[dstix.at[j0 + b]],
                                  ssems.at[b]).wait()
        return 0
    lax.fori_loop(0, NGRP, deg_group, 0)
    plsc.subcore_barrier()

    pltpu.sync_copy(acc_sh.at[pl.ds(base, RPT)], t_buf)
    pltpu.sync_copy(h_hbm.at[pl.ds(base, RPT)], c2_buf)

    def rsqrt(v):
        i = lax.bitcast_convert_type(v, jnp.int32)
        i = jnp.int32(0x5F3759DF) - lax.shift_right_logical(i, 1)
        y = lax.bitcast_convert_type(i, jnp.float32)
        half = 0.5 * v
        y = y * (1.5 - half * y * y)
        y = y * (1.5 - half * y * y)
        y = y * (1.5 - half * y * y)
        y = y * (1.5 - half * y * y)
        return y

    def init_body(j, _):
        y = rsqrt(t_buf[j] + 1.0)
        u0 = y * c2_buf[j]
        u_buf[j] = u0
        c2_buf[j] = ALPHA * u0
        c1_buf[j] = (1.0 - ALPHA) * y * y
        return 0
    lax.fori_loop(0, RPT, init_body, 0)

    pltpu.sync_copy(u_buf, u_sh.at[pl.ds(base, RPT)])
    pltpu.sync_copy(z_buf, acc_sh.at[pl.ds(base, RPT)])
    plsc.subcore_barrier()

    def step(_, carry):
        for b in range(NBUF):
            pltpu.async_copy(u_sh.at[srcix.at[b]], gbufs.at[b], gsems.at[b])

        def edge_group(jo, _c):
            j0 = jo * NBUF
            for b in range(NBUF):
                j = j0 + b
                pltpu.make_async_copy(u_sh.at[srcix.at[j]], gbufs.at[b],
                                      gsems.at[b]).wait()
                pltpu.async_copy(gbufs.at[b], acc_sh.at[dstix.at[j]],
                                 ssems.at[b], add=True)
            for b in range(NBUF):
                j = j0 + b
                pltpu.make_async_copy(gbufs.at[b], acc_sh.at[dstix.at[j]],
                                      ssems.at[b]).wait()

                @pl.when(j + NBUF < NCHUNK)
                def _():
                    pltpu.async_copy(u_sh.at[srcix.at[j + NBUF]],
                                     gbufs.at[b], gsems.at[b])
            return 0
        lax.fori_loop(0, NGRP, edge_group, 0)
        plsc.subcore_barrier()

        pltpu.sync_copy(acc_sh.at[pl.ds(base, RPT)], t_buf)

        def comb(j, _c):
            u_buf[j] = c1_buf[j] * (t_buf[j] + u_buf[j]) + c2_buf[j]
            return 0
        lax.fori_loop(0, RPT, comb, 0)

        pltpu.sync_copy(u_buf, u_sh.at[pl.ds(base, RPT)])
        pltpu.sync_copy(z_buf, acc_sh.at[pl.ds(base, RPT)])
        plsc.subcore_barrier()
        return carry
    lax.fori_loop(0, K, step, 0)

    def fin(j, _c):
        t_buf[j] = u_buf[j] * rsqrt(c1_buf[j] * (1.0 / (1.0 - ALPHA)))
        return 0
    lax.fori_loop(0, RPT, fin, 0)
    pltpu.sync_copy(t_buf, out_hbm.at[pl.ds(base, RPT)])


def kernel(x, edge_index, W1, W2):
    src = edge_index[0].astype(jnp.int32)
    dst = edge_index[1].astype(jnp.int32)
    pad = E_PAD - N_EDGES
    pad_idx = N_NODES + (jnp.arange(pad, dtype=jnp.int32) % 16)
    srcp = jnp.concatenate([src, pad_idx]).reshape(NS, NCHUNK, C)
    dstp = jnp.concatenate([dst, pad_idx]).reshape(NS, NCHUNK, C)

    xp = jnp.pad(x, ((0, NP - N_NODES), (0, 0)))
    h = _mlp(xp, W1.T, W2.T)
    z = _appnp_sc(srcp, dstp, h)
    return z[:N_NODES]


# speedup vs baseline: 70.4825x; 1.4094x over previous
import functools

import jax
import jax.numpy as jnp
from jax import lax
from jax.experimental import pallas as pl
from jax.experimental.pallas import tpu as pltpu
from jax.experimental.pallas import tpu_sc as plsc

N_NODES = 10000
N_EDGES = 320000
N_FEATURES = 128
N_HIDDEN = 64
N_CLASSES = 16
K = 10
ALPHA = 0.1

NC = 2
NS = 16
LANES = 16
C = 128
NBUF = 8
NCHUNK = 80
NGRP = NCHUNK // NBUF
EPT = NCHUNK * C
E_PAD = EPT * NS * NC
NP = 10240
HALF = NP // 2
RPW = NP // (NC * NS)
ZR = NP // NS
NPOLL = 64


def _mlp_body(x_ref, w1t_ref, w2t_ref, h_ref):
    h = jnp.maximum(jnp.dot(x_ref[...], w1t_ref[...],
                            preferred_element_type=jnp.float32), 0.0)
    h_ref[...] = jnp.dot(h, w2t_ref[...], preferred_element_type=jnp.float32)


def _mlp(xp, w1t, w2t):
    blk = NP // 4
    return pl.pallas_call(
        _mlp_body,
        grid=(4,),
        in_specs=[
            pl.BlockSpec((blk, N_FEATURES), lambda i: (i, 0)),
            pl.BlockSpec((N_FEATURES, N_HIDDEN), lambda i: (0, 0)),
            pl.BlockSpec((N_HIDDEN, N_CLASSES), lambda i: (0, 0)),
        ],
        out_specs=pl.BlockSpec((blk, N_CLASSES), lambda i: (i, 0)),
        out_shape=jax.ShapeDtypeStruct((NP, N_CLASSES), jnp.float32),
    )(xp, w1t, w2t)


_MESH = plsc.VectorSubcoreMesh(
    core_axis_name="c", subcore_axis_name="s", num_cores=NC)


@functools.partial(
    pl.kernel,
    out_type=(
        jax.ShapeDtypeStruct((NP, LANES), jnp.float32),
        jax.ShapeDtypeStruct((NC, HALF, LANES), jnp.float32),
        jax.ShapeDtypeStruct((NP, LANES), jnp.float32),
        jax.ShapeDtypeStruct((NC, 16), jnp.int32),
    ),
    mesh=_MESH,
    compiler_params=pltpu.CompilerParams(use_tc_tiling_on_sc=False),
    scratch_types=[
        pltpu.VMEM_SHARED((NP, LANES), jnp.float32),
        pltpu.VMEM_SHARED((NP, LANES), jnp.float32),
        pltpu.VMEM((NCHUNK, C), jnp.int32),
        pltpu.VMEM((NCHUNK, C), jnp.int32),
        pltpu.VMEM((RPW, LANES), jnp.float32),
        pltpu.VMEM((RPW, LANES), jnp.float32),
        pltpu.VMEM((RPW, LANES), jnp.float32),
        pltpu.VMEM((RPW, LANES), jnp.float32),
        pltpu.VMEM((RPW, LANES), jnp.float32),
        pltpu.VMEM((ZR, LANES), jnp.float32),
        pltpu.VMEM((NBUF, C, LANES), jnp.float32),
        pltpu.VMEM((16,), jnp.int32),
        pltpu.VMEM((16,), jnp.int32),
        pltpu.SMEM((16,), jnp.int32),
        pltpu.SemaphoreType.DMA((NBUF,)),
        pltpu.SemaphoreType.DMA((NBUF,)),
    ],
)
def _appnp_sc(src_hbm, dst_hbm, h_hbm, out_hbm, xacc, xu, flags,
              u_sh, acc_sh, srcix, dstix, u_buf, c1_buf, c2_buf, t_buf,
              x_buf, z_buf, gbufs, fv, fr, fs, gsems, ssems):
    t = lax.axis_index("s")
    cix = lax.axis_index("c")
    ocix = 1 - cix
    ones16 = jnp.full((16,), 1, jnp.int32)
    gb = pl.multiple_of(cix * HALF + t * RPW, 8)
    ob = pl.multiple_of(ocix * HALF + t * RPW, 8)

    @pl.when(t == 0)
    def _():
        fv[...] = ones16 * 0
        pltpu.sync_copy(fv, flags.at[cix])

    def xsync(seq):
        plsc.subcore_barrier()

        @pl.when(t == 0)
        def _():
            fv[...] = ones16 * seq
            pltpu.sync_copy(fv, flags.at[cix])

        def poll(i, done):
            @pl.when(done == 0)
            def _():
                pltpu.sync_copy(flags.at[ocix], fr)
            return jnp.where(fr[...][0] >= seq, jnp.int32(1), done)

        fr[...] = ones16 * jnp.int32(-1)
        lax.fori_loop(0, NPOLL, poll, jnp.int32(0))
        plsc.subcore_barrier()

    pltpu.sync_copy(src_hbm.at[cix, t], srcix)
    pltpu.sync_copy(dst_hbm.at[cix, t], dstix)

    def zfill(j, _):
        z_buf[j] = jnp.zeros((LANES,), jnp.float32)
        return 0
    lax.fori_loop(0, ZR, zfill, 0)

    def onesfill(j, _):
        gbufs[0, j] = jnp.full((LANES,), 1.0, jnp.float32)
        return 0
    lax.fori_loop(0, C, onesfill, 0)

    zb = pl.multiple_of(t * ZR, 8)
    pltpu.sync_copy(z_buf, u_sh.at[pl.ds(zb, ZR)])
    pltpu.sync_copy(z_buf, acc_sh.at[pl.ds(zb, ZR)])
    plsc.subcore_barrier()

    def deg_group(jo, _c):
        j0 = jo * NBUF
        for b in range(NBUF):
            pltpu.async_copy(gbufs.at[0], acc_sh.at[dstix.at[j0 + b]],
                             ssems.at[b], add=True)
        for b in range(NBUF):
            pltpu.make_async_copy(gbufs.at[0], acc_sh.at[dstix.at[j0 + b]],
                                  ssems.at[b]).wait()
        return 0
    lax.fori_loop(0, NGRP, deg_group, 0)
    plsc.subcore_barrier()

    pltpu.sync_copy(acc_sh.at[pl.ds(ob, RPW)], xacc.at[cix, pl.ds(t * RPW, RPW)])
    xsync(1)
    pltpu.sync_copy(acc_sh.at[pl.ds(gb, RPW)], t_buf)
    pltpu.sync_copy(xacc.at[ocix, pl.ds(t * RPW, RPW)], x_buf)
    pltpu.sync_copy(h_hbm.at[pl.ds(gb, RPW)], c2_buf)

    def rsqrt(v):
        i = lax.bitcast_convert_type(v, jnp.int32)
        i = jnp.int32(0x5F3759DF) - lax.shift_right_logical(i, 1)
        y = lax.bitcast_convert_type(i, jnp.float32)
        half = 0.5 * v
        y = y * (1.5 - half * y * y)
        y = y * (1.5 - half * y * y)
        y = y * (1.5 - half * y * y)
        y = y * (1.5 - half * y * y)
        return y

    def init_body(j, _):
        y = rsqrt(t_buf[j] + x_buf[j] + 1.0)
        u0 = y * c2_buf[j]
        u_buf[j] = u0
        c2_buf[j] = ALPHA * u0
        c1_buf[j] = (1.0 - ALPHA) * y * y
        return 0
    lax.fori_loop(0, RPW, init_body, 0)

    plsc.subcore_barrier()
    pltpu.sync_copy(z_buf, acc_sh.at[pl.ds(zb, ZR)])
    pltpu.sync_copy(u_buf, u_sh.at[pl.ds(gb, RPW)])
    pltpu.sync_copy(u_buf, xu.at[pl.ds(gb, RPW)])
    xsync(2)
    pltpu.sync_copy(xu.at[pl.ds(ob, RPW)], u_sh.at[pl.ds(ob, RPW)])
    plsc.subcore_barrier()

    def step(k, carry):
        for b in range(NBUF):
            pltpu.async_copy(u_sh.at[srcix.at[b]], gbufs.at[b], gsems.at[b])

        def edge_group(jo, _c):
            j0 = jo * NBUF
            for b in range(NBUF):
                j = j0 + b
                pltpu.make_async_copy(u_sh.at[srcix.at[j]], gbufs.at[b],
                                      gsems.at[b]).wait()
                pltpu.async_copy(gbufs.at[b], acc_sh.at[dstix.at[j]],
                                 ssems.at[b], add=True)
            for b in range(NBUF):
                j = j0 + b
                pltpu.make_async_copy(gbufs.at[b], acc_sh.at[dstix.at[j]],
                                      ssems.at[b]).wait()

                @pl.when(j + NBUF < NCHUNK)
                def _():
                    pltpu.async_copy(u_sh.at[srcix.at[j + NBUF]],
                                     gbufs.at[b], gsems.at[b])
            return 0
        lax.fori_loop(0, NGRP, edge_group, 0)
        plsc.subcore_barrier()

        pltpu.sync_copy(acc_sh.at[pl.ds(ob, RPW)],
                        xacc.at[cix, pl.ds(t * RPW, RPW)])
        xsync(3 + 2 * k)
        pltpu.sync_copy(acc_sh.at[pl.ds(gb, RPW)], t_buf)
        pltpu.sync_copy(xacc.at[ocix, pl.ds(t * RPW, RPW)], x_buf)

        def comb(j, _c):
            u_buf[j] = c1_buf[j] * (t_buf[j] + x_buf[j] + u_buf[j]) + c2_buf[j]
            return 0
        lax.fori_loop(0, RPW, comb, 0)

        plsc.subcore_barrier()
        pltpu.sync_copy(z_buf, acc_sh.at[pl.ds(zb, ZR)])
        pltpu.sync_copy(u_buf, u_sh.at[pl.ds(gb, RPW)])
        pltpu.sync_copy(u_buf, xu.at[pl.ds(gb, RPW)])
        xsync(4 + 2 * k)
        pltpu.sync_copy(xu.at[pl.ds(ob, RPW)], u_sh.at[pl.ds(ob, RPW)])
        plsc.subcore_barrier()
        return carry
    lax.fori_loop(0, K, step, 0)

    def fin(j, _c):
        t_buf[j] = u_buf[j] * rsqrt(c1_buf[j] * (1.0 / (1.0 - ALPHA)))
        return 0
    lax.fori_loop(0, RPW, fin, 0)
    pltpu.sync_copy(t_buf, out_hbm.at[pl.ds(gb, RPW)])


def kernel(x, edge_index, W1, W2):
    src = edge_index[0].astype(jnp.int32)
    dst = edge_index[1].astype(jnp.int32)
    pad = E_PAD - N_EDGES
    pad_idx = N_NODES + (jnp.arange(pad, dtype=jnp.int32) % 16)
    srcp = jnp.concatenate([src, pad_idx]).reshape(NC, NS, NCHUNK, C)
    dstp = jnp.concatenate([dst, pad_idx]).reshape(NC, NS, NCHUNK, C)

    xp = jnp.pad(x, ((0, NP - N_NODES), (0, 0)))
    h = _mlp(xp, W1.T, W2.T)
    z, _, _, _ = _appnp_sc(srcp, dstp, h)
    return z[:N_NODES]


# speedup vs baseline: 70.5218x; 1.0006x over previous
import functools

import jax
import jax.numpy as jnp
from jax import lax
from jax.experimental import pallas as pl
from jax.experimental.pallas import tpu as pltpu
from jax.experimental.pallas import tpu_sc as plsc

N_NODES = 10000
N_EDGES = 320000
N_FEATURES = 128
N_HIDDEN = 64
N_CLASSES = 16
K = 10
ALPHA = 0.1

NC = 2
NS = 16
LANES = 16
C = 128
NBUF = 8
NCHUNK = 80
NGRP = NCHUNK // NBUF
EPT = NCHUNK * C
E_PAD = EPT * NS * NC
NP = 10240
HALF = NP // 2
ZR = NP // NS
NPOLL = 64


def _mlp_body(x_ref, w1t_ref, w2t_ref, h_ref):
    h = jnp.maximum(jnp.dot(x_ref[...], w1t_ref[...],
                            preferred_element_type=jnp.float32), 0.0)
    h_ref[...] = jnp.dot(h, w2t_ref[...], preferred_element_type=jnp.float32)


def _mlp(xp, w1t, w2t):
    blk = NP // 4
    return pl.pallas_call(
        _mlp_body,
        grid=(4,),
        in_specs=[
            pl.BlockSpec((blk, N_FEATURES), lambda i: (i, 0)),
            pl.BlockSpec((N_FEATURES, N_HIDDEN), lambda i: (0, 0)),
            pl.BlockSpec((N_HIDDEN, N_CLASSES), lambda i: (0, 0)),
        ],
        out_specs=pl.BlockSpec((blk, N_CLASSES), lambda i: (i, 0)),
        out_shape=jax.ShapeDtypeStruct((NP, N_CLASSES), jnp.float32),
    )(xp, w1t, w2t)


_MESH = plsc.VectorSubcoreMesh(
    core_axis_name="c", subcore_axis_name="s", num_cores=NC)


@functools.partial(
    pl.kernel,
    out_type=(
        jax.ShapeDtypeStruct((NP, LANES), jnp.float32),
        jax.ShapeDtypeStruct((NC, NP, LANES), jnp.float32),
        jax.ShapeDtypeStruct((NC, 16), jnp.int32),
    ),
    mesh=_MESH,
    compiler_params=pltpu.CompilerParams(use_tc_tiling_on_sc=False),
    scratch_types=[
        pltpu.VMEM_SHARED((NP, LANES), jnp.float32),
        pltpu.VMEM_SHARED((NP, LANES), jnp.float32),
        pltpu.VMEM((NCHUNK, C), jnp.int32),
        pltpu.VMEM((NCHUNK, C), jnp.int32),
        pltpu.VMEM((ZR, LANES), jnp.float32),
        pltpu.VMEM((ZR, LANES), jnp.float32),
        pltpu.VMEM((ZR, LANES), jnp.float32),
        pltpu.VMEM((ZR, LANES), jnp.float32),
        pltpu.VMEM((ZR, LANES), jnp.float32),
        pltpu.VMEM((ZR, LANES), jnp.float32),
        pltpu.VMEM((NBUF, C, LANES), jnp.float32),
        pltpu.VMEM((16,), jnp.int32),
        pltpu.VMEM((16,), jnp.int32),
        pltpu.SemaphoreType.DMA((NBUF,)),
        pltpu.SemaphoreType.DMA((NBUF,)),
    ],
)
def _appnp_sc(src_hbm, dst_hbm, h_hbm, out_hbm, xacc, flags,
              u_sh, acc_sh, srcix, dstix, u_buf, c1_buf, c2_buf, t_buf,
              x_buf, z_buf, gbufs, fv, fr, gsems, ssems):
    t = lax.axis_index("s")
    cix = lax.axis_index("c")
    ocix = 1 - cix
    ones16 = jnp.full((16,), 1, jnp.int32)
    rb = pl.multiple_of(t * ZR, 8)

    @pl.when(t == 0)
    def _():
        fv[...] = ones16 * 0
        pltpu.sync_copy(fv, flags.at[cix])

    def xsync(seq):
        plsc.subcore_barrier()

        @pl.when(t == 0)
        def _():
            fv[...] = ones16 * seq
            pltpu.sync_copy(fv, flags.at[cix])

        def poll(i, done):
            @pl.when(done == 0)
            def _():
                pltpu.sync_copy(flags.at[ocix], fr)
            return jnp.where(fr[...][0] >= seq, jnp.int32(1), done)

        fr[...] = ones16 * jnp.int32(-1)
        lax.fori_loop(0, NPOLL, poll, jnp.int32(0))
        plsc.subcore_barrier()

    pltpu.sync_copy(src_hbm.at[cix, t], srcix)
    pltpu.sync_copy(dst_hbm.at[cix, t], dstix)

    def zfill(j, _):
        z_buf[j] = jnp.zeros((LANES,), jnp.float32)
        return 0
    lax.fori_loop(0, ZR, zfill, 0)

    def onesfill(j, _):
        gbufs[0, j] = jnp.full((LANES,), 1.0, jnp.float32)
        return 0
    lax.fori_loop(0, C, onesfill, 0)

    pltpu.sync_copy(z_buf, u_sh.at[pl.ds(rb, ZR)])
    pltpu.sync_copy(z_buf, acc_sh.at[pl.ds(rb, ZR)])
    plsc.subcore_barrier()

    def deg_group(jo, _c):
        j0 = jo * NBUF
        for b in range(NBUF):
            pltpu.async_copy(gbufs.at[0], acc_sh.at[dstix.at[j0 + b]],
                             ssems.at[b], add=True)
        for b in range(NBUF):
            pltpu.make_async_copy(gbufs.at[0], acc_sh.at[dstix.at[j0 + b]],
                                  ssems.at[b]).wait()
        return 0
    lax.fori_loop(0, NGRP, deg_group, 0)
    plsc.subcore_barrier()

    pltpu.sync_copy(acc_sh.at[pl.ds(rb, ZR)], xacc.at[cix, pl.ds(rb, ZR)])
    xsync(1)
    pltpu.sync_copy(acc_sh.at[pl.ds(rb, ZR)], t_buf)
    pltpu.sync_copy(xacc.at[ocix, pl.ds(rb, ZR)], x_buf)
    pltpu.sync_copy(h_hbm.at[pl.ds(rb, ZR)], c2_buf)

    def rsqrt(v):
        i = lax.bitcast_convert_type(v, jnp.int32)
        i = jnp.int32(0x5F3759DF) - lax.shift_right_logical(i, 1)
        y = lax.bitcast_convert_type(i, jnp.float32)
        half = 0.5 * v
        y = y * (1.5 - half * y * y)
        y = y * (1.5 - half * y * y)
        y = y * (1.5 - half * y * y)
        y = y * (1.5 - half * y * y)
        return y

    def init_body(j, _):
        y = rsqrt(t_buf[j] + x_buf[j] + 1.0)
        u0 = y * c2_buf[j]
        u_buf[j] = u0
        c2_buf[j] = ALPHA * u0
        c1_buf[j] = (1.0 - ALPHA) * y * y
        return 0
    lax.fori_loop(0, ZR, init_body, 0)

    pltpu.sync_copy(z_buf, acc_sh.at[pl.ds(rb, ZR)])
    pltpu.sync_copy(u_buf, u_sh.at[pl.ds(rb, ZR)])
    plsc.subcore_barrier()

    def step(k, carry):
        for b in range(NBUF):
            pltpu.async_copy(u_sh.at[srcix.at[b]], gbufs.at[b], gsems.at[b])

        def edge_group(jo, _c):
            j0 = jo * NBUF
            for b in range(NBUF):
                j = j0 + b
                pltpu.make_async_copy(u_sh.at[srcix.at[j]], gbufs.at[b],
                                      gsems.at[b]).wait()
                pltpu.async_copy(gbufs.at[b], acc_sh.at[dstix.at[j]],
                                 ssems.at[b], add=True)
            for b in range(NBUF):
                j = j0 + b
                pltpu.make_async_copy(gbufs.at[b], acc_sh.at[dstix.at[j]],
                                      ssems.at[b]).wait()

                @pl.when(j + NBUF < NCHUNK)
                def _():
                    pltpu.async_copy(u_sh.at[srcix.at[j + NBUF]],
                                     gbufs.at[b], gsems.at[b])
            return 0
        lax.fori_loop(0, NGRP, edge_group, 0)
        plsc.subcore_barrier()

        pltpu.sync_copy(acc_sh.at[pl.ds(rb, ZR)], xacc.at[cix, pl.ds(rb, ZR)])
        xsync(2 + k)
        pltpu.sync_copy(acc_sh.at[pl.ds(rb, ZR)], t_buf)
        pltpu.sync_copy(xacc.at[ocix, pl.ds(rb, ZR)], x_buf)

        def comb(j, _c):
            u_buf[j] = c1_buf[j] * (t_buf[j] + x_buf[j] + u_buf[j]) + c2_buf[j]
            return 0
        lax.fori_loop(0, ZR, comb, 0)

        pltpu.sync_copy(z_buf, acc_sh.at[pl.ds(rb, ZR)])
        pltpu.sync_copy(u_buf, u_sh.at[pl.ds(rb, ZR)])
        plsc.subcore_barrier()
        return carry
    lax.fori_loop(0, K, step, 0)

    @pl.when(lax.div(t, 8) == cix)
    def _():
        def fin(j, _c):
            t_buf[j] = u_buf[j] * rsqrt(c1_buf[j] * (1.0 / (1.0 - ALPHA)))
            return 0
        lax.fori_loop(0, ZR, fin, 0)
        pltpu.sync_copy(t_buf, out_hbm.at[pl.ds(rb, ZR)])


def kernel(x, edge_index, W1, W2):
    src = edge_index[0].astype(jnp.int32)
    dst = edge_index[1].astype(jnp.int32)
    pad = E_PAD - N_EDGES
    pad_idx = N_NODES + (jnp.arange(pad, dtype=jnp.int32) % 16)
    srcp = jnp.concatenate([src, pad_idx]).reshape(NC, NS, NCHUNK, C)
    dstp = jnp.concatenate([dst, pad_idx]).reshape(NC, NS, NCHUNK, C)

    xp = jnp.pad(x, ((0, NP - N_NODES), (0, 0)))
    h = _mlp(xp, W1.T, W2.T)
    z, _, _ = _appnp_sc(srcp, dstp, h)
    return z[:N_NODES]


# speedup vs baseline: 76.0637x; 1.0786x over previous
import functools

import jax
import jax.numpy as jnp
from jax import lax
from jax.experimental import pallas as pl
from jax.experimental.pallas import tpu as pltpu
from jax.experimental.pallas import tpu_sc as plsc

N_NODES = 10000
N_EDGES = 320000
N_FEATURES = 128
N_HIDDEN = 64
N_CLASSES = 16
K = 10
ALPHA = 0.1

NC = 2
NS = 16
LANES = 16
C = 128
NBUF = 8
NCHUNK = 80
NGRP = NCHUNK // NBUF
EPT = NCHUNK * C
E_PAD = EPT * NS * NC
NP = 10240
HALF = NP // 2
ZR = NP // NS
NPOLL = 64


def _mlp_body(x_ref, w1t_ref, w2t_ref, h_ref):
    h = jnp.maximum(jnp.dot(x_ref[...], w1t_ref[...],
                            preferred_element_type=jnp.float32), 0.0)
    h_ref[...] = jnp.dot(h, w2t_ref[...], preferred_element_type=jnp.float32)


def _mlp(xp, w1t, w2t):
    blk = NP // 4
    return pl.pallas_call(
        _mlp_body,
        grid=(4,),
        in_specs=[
            pl.BlockSpec((blk, N_FEATURES), lambda i: (i, 0)),
            pl.BlockSpec((N_FEATURES, N_HIDDEN), lambda i: (0, 0)),
            pl.BlockSpec((N_HIDDEN, N_CLASSES), lambda i: (0, 0)),
        ],
        out_specs=pl.BlockSpec((blk, N_CLASSES), lambda i: (i, 0)),
        out_shape=jax.ShapeDtypeStruct((NP, N_CLASSES), jnp.float32),
    )(xp, w1t, w2t)


_MESH = plsc.VectorSubcoreMesh(
    core_axis_name="c", subcore_axis_name="s", num_cores=NC)


@functools.partial(
    pl.kernel,
    out_type=(
        jax.ShapeDtypeStruct((NP, LANES), jnp.float32),
        jax.ShapeDtypeStruct((NC, NP, LANES), jnp.float32),
        jax.ShapeDtypeStruct((NC, 16), jnp.int32),
    ),
    mesh=_MESH,
    compiler_params=pltpu.CompilerParams(use_tc_tiling_on_sc=False),
    scratch_types=[
        pltpu.VMEM_SHARED((NP, LANES), jnp.float32),
        pltpu.VMEM_SHARED((NP, LANES), jnp.float32),
        pltpu.VMEM((NCHUNK, C), jnp.int32),
        pltpu.VMEM((NCHUNK, C), jnp.int32),
        pltpu.VMEM((ZR, LANES), jnp.float32),
        pltpu.VMEM((ZR, LANES), jnp.float32),
        pltpu.VMEM((ZR, LANES), jnp.float32),
        pltpu.VMEM((ZR, LANES), jnp.float32),
        pltpu.VMEM((ZR, LANES), jnp.float32),
        pltpu.VMEM((ZR, LANES), jnp.float32),
        pltpu.VMEM((NBUF, C, LANES), jnp.float32),
        pltpu.VMEM((16,), jnp.int32),
        pltpu.VMEM((16,), jnp.int32),
        pltpu.SemaphoreType.DMA((NBUF,)),
        pltpu.SemaphoreType.DMA((NBUF,)),
    ],
)
def _appnp_sc(src_hbm, dst_hbm, h_hbm, out_hbm, xacc, flags,
              u_sh, acc_sh, srcix, dstix, u_buf, c1_buf, c2_buf, t_buf,
              x_buf, z_buf, gbufs, fv, fr, gsems, ssems):
    t = lax.axis_index("s")
    cix = lax.axis_index("c")
    ocix = 1 - cix
    ones16 = jnp.full((16,), 1, jnp.int32)
    rb = pl.multiple_of(t * ZR, 8)

    @pl.when(t == 0)
    def _():
        fv[...] = ones16 * 0
        pltpu.sync_copy(fv, flags.at[cix])

    def xsync(seq):
        plsc.subcore_barrier()

        @pl.when(t == 0)
        def _():
            fv[...] = ones16 * seq
            pltpu.sync_copy(fv, flags.at[cix])

        def poll(i, done):
            @pl.when(done == 0)
            def _():
                pltpu.sync_copy(flags.at[ocix], fr)
            return jnp.where(fr[...][0] >= seq, jnp.int32(1), done)

        fr[...] = ones16 * jnp.int32(-1)
        lax.fori_loop(0, NPOLL, poll, jnp.int32(0))
        plsc.subcore_barrier()

    pltpu.sync_copy(src_hbm.at[cix, t], srcix)
    pltpu.sync_copy(dst_hbm.at[cix, t], dstix)
    pltpu.sync_copy(h_hbm.at[pl.ds(rb, ZR)], c2_buf)

    @plsc.parallel_loop(0, ZR, 1, unroll=8)
    def zfill(j):
        z_buf[j] = jnp.zeros((LANES,), jnp.float32)

    @plsc.parallel_loop(0, C, 1, unroll=8)
    def onesfill(j):
        gbufs[0, j] = jnp.full((LANES,), 1.0, jnp.float32)

    pltpu.sync_copy(z_buf, u_sh.at[pl.ds(rb, ZR)])
    pltpu.sync_copy(z_buf, acc_sh.at[pl.ds(rb, ZR)])
    plsc.subcore_barrier()

    def deg_group(jo, _c):
        j0 = jo * NBUF
        for b in range(NBUF):
            pltpu.async_copy(gbufs.at[0], acc_sh.at[dstix.at[j0 + b]],
                             ssems.at[b], add=True)
        for b in range(NBUF):
            pltpu.make_async_copy(gbufs.at[0], acc_sh.at[dstix.at[j0 + b]],
                                  ssems.at[b]).wait()
        return 0
    lax.fori_loop(0, NGRP, deg_group, 0)
    plsc.subcore_barrier()

    pltpu.sync_copy(acc_sh.at[pl.ds(rb, ZR)], xacc.at[cix, pl.ds(rb, ZR)])
    pltpu.sync_copy(acc_sh.at[pl.ds(rb, ZR)], t_buf)
    pltpu.sync_copy(z_buf, acc_sh.at[pl.ds(rb, ZR)])
    xsync(1)
    pltpu.sync_copy(xacc.at[ocix, pl.ds(rb, ZR)], x_buf)

    def rsqrt(v):
        i = lax.bitcast_convert_type(v, jnp.int32)
        i = jnp.int32(0x5F3759DF) - lax.shift_right_logical(i, 1)
        y = lax.bitcast_convert_type(i, jnp.float32)
        half = 0.5 * v
        y = y * (1.5 - half * y * y)
        y = y * (1.5 - half * y * y)
        y = y * (1.5 - half * y * y)
        y = y * (1.5 - half * y * y)
        return y

    @plsc.parallel_loop(0, ZR, 1, unroll=8)
    def init_body(j):
        y = rsqrt(t_buf[j] + x_buf[j] + 1.0)
        u0 = y * c2_buf[j]
        u_buf[j] = u0
        c2_buf[j] = ALPHA * u0
        c1_buf[j] = (1.0 - ALPHA) * y * y

    pltpu.sync_copy(u_buf, u_sh.at[pl.ds(rb, ZR)])
    plsc.subcore_barrier()

    def step(k, carry):
        for b in range(NBUF):
            pltpu.async_copy(u_sh.at[srcix.at[b]], gbufs.at[b], gsems.at[b])

        def edge_group(jo, _c):
            j0 = jo * NBUF
            for b in range(NBUF):
                j = j0 + b
                pltpu.make_async_copy(u_sh.at[srcix.at[j]], gbufs.at[b],
                                      gsems.at[b]).wait()
                pltpu.async_copy(gbufs.at[b], acc_sh.at[dstix.at[j]],
                                 ssems.at[b], add=True)
            for b in range(NBUF):
                j = j0 + b
                pltpu.make_async_copy(gbufs.at[b], acc_sh.at[dstix.at[j]],
                                      ssems.at[b]).wait()

                @pl.when(j + NBUF < NCHUNK)
                def _():
                    pltpu.async_copy(u_sh.at[srcix.at[j + NBUF]],
                                     gbufs.at[b], gsems.at[b])
            return 0
        lax.fori_loop(0, NGRP, edge_group, 0)
        plsc.subcore_barrier()

        pltpu.sync_copy(acc_sh.at[pl.ds(rb, ZR)], xacc.at[cix, pl.ds(rb, ZR)])
        pltpu.sync_copy(acc_sh.at[pl.ds(rb, ZR)], t_buf)
        pltpu.sync_copy(z_buf, acc_sh.at[pl.ds(rb, ZR)])
        xsync(2 + k)
        pltpu.sync_copy(xacc.at[ocix, pl.ds(rb, ZR)], x_buf)

        @plsc.parallel_loop(0, ZR, 1, unroll=8)
        def comb(j):
            u_buf[j] = c1_buf[j] * (t_buf[j] + x_buf[j] + u_buf[j]) + c2_buf[j]

        pltpu.sync_copy(u_buf, u_sh.at[pl.ds(rb, ZR)])
        plsc.subcore_barrier()
        return carry
    lax.fori_loop(0, K, step, 0)

    @pl.when(lax.div(t, 8) == cix)
    def _():
        @plsc.parallel_loop(0, ZR, 1, unroll=8)
        def fin(j):
            t_buf[j] = u_buf[j] * rsqrt(c1_buf[j] * (1.0 / (1.0 - ALPHA)))

        pltpu.sync_copy(t_buf, out_hbm.at[pl.ds(rb, ZR)])


def kernel(x, edge_index, W1, W2):
    src = edge_index[0].astype(jnp.int32)
    dst = edge_index[1].astype(jnp.int32)
    pad = E_PAD - N_EDGES
    pad_idx = N_NODES + (jnp.arange(pad, dtype=jnp.int32) % 16)
    srcp = jnp.concatenate([src, pad_idx]).reshape(NC, NS, NCHUNK, C)
    dstp = jnp.concatenate([dst, pad_idx]).reshape(NC, NS, NCHUNK, C)

    xp = jnp.pad(x, ((0, NP - N_NODES), (0, 0)))
    h = _mlp(xp, W1.T, W2.T)
    z, _, _ = _appnp_sc(srcp, dstp, h)
    return z[:N_NODES]
